# baseline (speedup 1.0000x reference)
import functools

import jax
import jax.numpy as jnp
from jax import lax
from jax.experimental import pallas as pl
from jax.experimental.pallas import tpu as pltpu
from jax.experimental.pallas import tpu_sc as plsc

NC = 2
NS = 16
NW = NC * NS
L = 16
BB = 128


@functools.lru_cache(maxsize=None)
def _build(batch, hist, num_emb, emb_dim, rh):
    nchunk = hist // rh
    assert nchunk * rh == hist and batch == NW * BB

    def body(xt_hbm, table_hbm, out_hbm, table_v,
             idx0, idx1, rows0, rows1, isem0, isem1, osem0, osem1):
        wid = lax.axis_index("s") * NC + lax.axis_index("c")
        b0 = wid * BB
        q = wid // 2
        s0 = (wid % 2) * emb_dim

        idx = [idx0, idx1]
        rows = [rows0, rows1]
        isem = [isem0, isem1]
        osem = [osem0, osem1]

        in_cp = {}
        out_cp = {}
        in_cp[0] = pltpu.async_copy(
            xt_hbm.at[pl.ds(0, rh), pl.ds(b0, BB)], idx[0], isem[0])
        pltpu.sync_copy(table_hbm, table_v)
        for c in range(nchunk):
            cur = c % 2
            if c + 1 < nchunk:
                in_cp[c + 1] = pltpu.async_copy(
                    xt_hbm.at[pl.ds((c + 1) * rh, rh), pl.ds(b0, BB)],
                    idx[1 - cur], isem[1 - cur])
            in_cp[c].wait()
            if c >= 2:
                out_cp[c - 2].wait()

            @plsc.parallel_loop(0, rh, unroll=4)
            def row_body(r):
                for cv in range(BB // L):
                    xv = idx[cur][r, pl.ds(cv * L, L)] * emb_dim
                    for d in range(emb_dim):
                        rows[cur][r, d, pl.ds(cv * L, L)] = plsc.load_gather(
                            table_v, [xv + d])

            out_cp[c] = pltpu.async_copy(
                rows[cur],
                out_hbm.at[pl.ds(c * rh, rh), q, pl.ds(s0, emb_dim)],
                osem[cur])
        out_cp[nchunk - 2].wait()
        out_cp[nchunk - 1].wait()

    return pl.kernel(
        body,
        out_type=jax.ShapeDtypeStruct(
            (hist, NW // 2, 2 * emb_dim, BB), jnp.float32),
        mesh=plsc.VectorSubcoreMesh(core_axis_name="c", subcore_axis_name="s"),
        compiler_params=pltpu.CompilerParams(needs_layout_passes=False),
        scratch_types=[
            pltpu.VMEM((num_emb * emb_dim,), jnp.float32),
            pltpu.VMEM((rh, BB), jnp.int32),
            pltpu.VMEM((rh, BB), jnp.int32),
            pltpu.VMEM((rh, emb_dim, BB), jnp.float32),
            pltpu.VMEM((rh, emb_dim, BB), jnp.float32),
            pltpu.SemaphoreType.DMA,
            pltpu.SemaphoreType.DMA,
            pltpu.SemaphoreType.DMA,
            pltpu.SemaphoreType.DMA,
        ],
    )


def kernel(x, table):
    batch, hist = x.shape
    num_emb, emb_dim = table.shape
    fn = _build(batch, hist, num_emb, emb_dim, 8)
    out4 = fn(x.T.astype(jnp.int32), table.reshape(num_emb * emb_dim))
    out = out4.reshape(hist, NW, emb_dim, BB).transpose(1, 3, 0, 2)
    return out.reshape(batch, hist, emb_dim)


# speedup vs baseline: 1.1876x; 1.1876x over previous
import functools

import jax
import jax.numpy as jnp
from jax import lax
from jax.experimental import pallas as pl
from jax.experimental.pallas import tpu as pltpu
from jax.experimental.pallas import tpu_sc as plsc

NC = 2
NS = 16
NW = NC * NS
L = 16
BB = 128


@functools.lru_cache(maxsize=None)
def _build(batch, hist, num_emb, emb_dim, rh):
    nchunk = hist // rh
    assert nchunk * rh == hist and batch == NW * BB

    def body(xt_hbm, table_hbm, out_hbm, table_v,
             idx0, idx1, rows0, rows1, isem0, isem1, osem0, osem1):
        wid = lax.axis_index("s") * NC + lax.axis_index("c")
        b0 = wid * BB
        q = wid // 2
        s0 = (wid % 2) * emb_dim

        idx = [idx0, idx1]
        rows = [rows0, rows1]
        isem = [isem0, isem1]
        osem = [osem0, osem1]

        in_cp = {}
        out_cp = {}
        in_cp[0] = pltpu.async_copy(
            xt_hbm.at[pl.ds(0, rh), pl.ds(b0, BB)], idx[0], isem[0])
        pltpu.sync_copy(table_hbm, table_v)
        for c in range(nchunk):
            cur = c % 2
            if c + 1 < nchunk:
                in_cp[c + 1] = pltpu.async_copy(
                    xt_hbm.at[pl.ds((c + 1) * rh, rh), pl.ds(b0, BB)],
                    idx[1 - cur], isem[1 - cur])
            in_cp[c].wait()
            if c >= 2:
                out_cp[c - 2].wait()

            @plsc.parallel_loop(0, rh, unroll=8)
            def row_body(r):
                for cv in range(BB // L):
                    xv = idx[cur][r, pl.ds(cv * L, L)] * emb_dim
                    for d in range(emb_dim):
                        rows[cur][r, d, pl.ds(cv * L, L)] = plsc.load_gather(
                            table_v, [xv + d])

            out_cp[c] = pltpu.async_copy(
                rows[cur],
                out_hbm.at[pl.ds(c * rh, rh), q, pl.ds(s0, emb_dim)],
                osem[cur])
        out_cp[nchunk - 2].wait()
        out_cp[nchunk - 1].wait()

    return pl.kernel(
        body,
        out_type=jax.ShapeDtypeStruct(
            (hist, NW // 2, 2 * emb_dim, BB), jnp.float32),
        mesh=plsc.VectorSubcoreMesh(core_axis_name="c", subcore_axis_name="s"),
        compiler_params=pltpu.CompilerParams(needs_layout_passes=False),
        scratch_types=[
            pltpu.VMEM((num_emb * emb_dim,), jnp.float32),
            pltpu.VMEM((rh, BB), jnp.int32),
            pltpu.VMEM((rh, BB), jnp.int32),
            pltpu.VMEM((rh, emb_dim, BB), jnp.float32),
            pltpu.VMEM((rh, emb_dim, BB), jnp.float32),
            pltpu.SemaphoreType.DMA,
            pltpu.SemaphoreType.DMA,
            pltpu.SemaphoreType.DMA,
            pltpu.SemaphoreType.DMA,
        ],
    )


def kernel(x, table):
    batch, hist = x.shape
    num_emb, emb_dim = table.shape
    fn = _build(batch, hist, num_emb, emb_dim, 40)
    out4 = fn(x.T.astype(jnp.int32), table.reshape(num_emb * emb_dim))
    out = out4.reshape(hist, NW, emb_dim, BB).transpose(1, 3, 0, 2)
    return out.reshape(batch, hist, emb_dim)


# speedup vs baseline: 1.2419x; 1.0457x over previous
import functools

import jax
import jax.numpy as jnp
from jax import lax
from jax.experimental import pallas as pl
from jax.experimental.pallas import tpu as pltpu
from jax.experimental.pallas import tpu_sc as plsc

NC = 2
NS = 16
NW = NC * NS
L = 16
BB = 128


@functools.lru_cache(maxsize=None)
def _build(batch, hist, num_emb, emb_dim, rh):
    nchunk = hist // rh
    assert nchunk * rh == hist and batch == NW * BB

    def body(xt_hbm, table_hbm, out_hbm, table_v,
             idx0, idx1, rows0, rows1, isem0, isem1, osem0, osem1):
        wid = lax.axis_index("s") * NC + lax.axis_index("c")
        b0 = wid * BB
        q = wid // 2
        s0 = (wid % 2) * emb_dim

        idx = [idx0, idx1]
        rows = [rows0, rows1]
        isem = [isem0, isem1]
        osem = [osem0, osem1]

        in_cp = {}
        out_cp = {}
        in_cp[0] = pltpu.async_copy(
            xt_hbm.at[pl.ds(0, rh), pl.ds(b0, BB)], idx[0], isem[0])
        pltpu.sync_copy(table_hbm, table_v)
        for c in range(nchunk):
            cur = c % 2
            if c + 1 < nchunk:
                in_cp[c + 1] = pltpu.async_copy(
                    xt_hbm.at[pl.ds((c + 1) * rh, rh), pl.ds(b0, BB)],
                    idx[1 - cur], isem[1 - cur])
            in_cp[c].wait()
            if c >= 2:
                out_cp[c - 2].wait()

            @plsc.parallel_loop(0, rh, unroll=4)
            def row_body(r):
                for cv in range(BB // L):
                    xv = idx[cur][r, pl.ds(cv * L, L)] * emb_dim
                    for d in range(emb_dim):
                        rows[cur][r, d, pl.ds(cv * L, L)] = plsc.load_gather(
                            table_v, [xv + d])

            out_cp[c] = pltpu.async_copy(
                rows[cur],
                out_hbm.at[pl.ds(c * rh, rh), q, pl.ds(s0, emb_dim)],
                osem[cur])
        out_cp[nchunk - 2].wait()
        out_cp[nchunk - 1].wait()

    return pl.kernel(
        body,
        out_type=jax.ShapeDtypeStruct(
            (hist, NW // 2, 2 * emb_dim, BB), jnp.float32),
        mesh=plsc.VectorSubcoreMesh(core_axis_name="c", subcore_axis_name="s"),
        compiler_params=pltpu.CompilerParams(needs_layout_passes=False),
        scratch_types=[
            pltpu.VMEM((num_emb * emb_dim,), jnp.float32),
            pltpu.VMEM((rh, BB), jnp.int32),
            pltpu.VMEM((rh, BB), jnp.int32),
            pltpu.VMEM((rh, emb_dim, BB), jnp.float32),
            pltpu.VMEM((rh, emb_dim, BB), jnp.float32),
            pltpu.SemaphoreType.DMA,
            pltpu.SemaphoreType.DMA,
            pltpu.SemaphoreType.DMA,
            pltpu.SemaphoreType.DMA,
        ],
    )


def kernel(x, table):
    batch, hist = x.shape
    num_emb, emb_dim = table.shape
    fn = _build(batch, hist, num_emb, emb_dim, 40)
    out4 = fn(x.T.astype(jnp.int32), table.reshape(num_emb * emb_dim))
    out = out4.reshape(hist, NW, emb_dim, BB).transpose(1, 3, 0, 2)
    return out.reshape(batch, hist, emb_dim)


# speedup vs baseline: 1.2681x; 1.0211x over previous
import functools

import jax
import jax.numpy as jnp
from jax import lax
from jax.experimental import pallas as pl
from jax.experimental.pallas import tpu as pltpu
from jax.experimental.pallas import tpu_sc as plsc

NC = 2
NS = 16
NW = NC * NS
L = 16
BB = 128


@functools.lru_cache(maxsize=None)
def _build(batch, hist, num_emb, emb_dim, chunks):
    assert sum(chunks) == hist and batch == NW * BB
    nchunk = len(chunks)
    starts = [sum(chunks[:i]) for i in range(nchunk)]
    rh_max = max(chunks)

    def body(xt_hbm, table_hbm, out_hbm, table_v,
             idx0, idx1, rows0, rows1, isem0, isem1, osem0, osem1, tsem):
        wid = lax.axis_index("s") * NC + lax.axis_index("c")
        b0 = wid * BB
        q = wid // 2
        s0 = (wid % 2) * emb_dim

        idx = [idx0, idx1]
        rows = [rows0, rows1]
        isem = [isem0, isem1]
        osem = [osem0, osem1]

        in_cp = {}
        out_cp = {}
        in_cp[0] = pltpu.async_copy(
            xt_hbm.at[pl.ds(0, chunks[0]), pl.ds(b0, BB)],
            idx[0].at[pl.ds(0, chunks[0])], isem[0])
        t_cp = pltpu.async_copy(table_hbm, table_v, tsem)
        t_cp.wait()
        for c in range(nchunk):
            cur = c % 2
            rh = chunks[c]
            if c + 1 < nchunk:
                in_cp[c + 1] = pltpu.async_copy(
                    xt_hbm.at[pl.ds(starts[c + 1], chunks[c + 1]),
                              pl.ds(b0, BB)],
                    idx[1 - cur].at[pl.ds(0, chunks[c + 1])], isem[1 - cur])
            in_cp[c].wait()
            if c >= 2:
                out_cp[c - 2].wait()

            @plsc.parallel_loop(0, rh, unroll=4)
            def row_body(r):
                for cv in range(BB // L):
                    xv = idx[cur][r, pl.ds(cv * L, L)] * emb_dim
                    for d in range(emb_dim):
                        rows[cur][r, d, pl.ds(cv * L, L)] = plsc.load_gather(
                            table_v, [xv + d])

            out_cp[c] = pltpu.async_copy(
                rows[cur].at[pl.ds(0, rh)],
                out_hbm.at[pl.ds(starts[c], rh), q, pl.ds(s0, emb_dim)],
                osem[cur])
        out_cp[nchunk - 2].wait()
        out_cp[nchunk - 1].wait()

    return pl.kernel(
        body,
        out_type=jax.ShapeDtypeStruct(
            (hist, NW // 2, 2 * emb_dim, BB), jnp.float32),
        mesh=plsc.VectorSubcoreMesh(core_axis_name="c", subcore_axis_name="s"),
        compiler_params=pltpu.CompilerParams(needs_layout_passes=False),
        scratch_types=[
            pltpu.VMEM((num_emb * emb_dim,), jnp.float32),
            pltpu.VMEM((rh_max, BB), jnp.int32),
            pltpu.VMEM((rh_max, BB), jnp.int32),
            pltpu.VMEM((rh_max, emb_dim, BB), jnp.float32),
            pltpu.VMEM((rh_max, emb_dim, BB), jnp.float32),
            pltpu.SemaphoreType.DMA,
            pltpu.SemaphoreType.DMA,
            pltpu.SemaphoreType.DMA,
            pltpu.SemaphoreType.DMA,
            pltpu.SemaphoreType.DMA,
        ],
    )


def kernel(x, table):
    batch, hist = x.shape
    num_emb, emb_dim = table.shape
    fn = _build(batch, hist, num_emb, emb_dim, (8, 48, 48, 48, 48))
    out4 = fn(x.T.astype(jnp.int32), table.reshape(num_emb * emb_dim))
    out = out4.reshape(hist, NW, emb_dim, BB).transpose(1, 3, 0, 2)
    return out.reshape(batch, hist, emb_dim)
